# baseline (speedup 1.0000x reference)
import jax
import jax.numpy as jnp
from jax.experimental import pallas as pl
from jax.experimental.pallas import tpu as pltpu

H = 32
W = 32
C = 512
WP = W + 2
HP = H + 4
FLAT = HP * WP
ROWS = H * WP
BASE = 2 * WP


def _rpn_kernel(x_ref, wf_ref, bs_ref, wc_ref, bc_ref, wd_ref, bd_ref,
                probs_ref, deltas_ref, xpad_ref):
    xpad_ref[...] = jnp.zeros((FLAT, C), dtype=jnp.bfloat16)
    x = x_ref[0].astype(jnp.bfloat16)
    for y in range(H):
        dst = (y + 2) * WP + 1
        xpad_ref[dst:dst + W, :] = x[y * W:(y + 1) * W, :]

    xp = xpad_ref[...]
    wf = wf_ref[...].astype(jnp.bfloat16)
    acc = jnp.zeros((ROWS, C), dtype=jnp.float32)
    for t in range(9):
        dy, dx = t // 3 - 1, t % 3 - 1
        start = BASE + dy * WP + dx
        acc = acc + jnp.dot(xp[start:start + ROWS, :], wf[t],
                            preferred_element_type=jnp.float32)
    shared = jnp.maximum(acc + bs_ref[0], 0.0)

    cls = jnp.dot(shared, wc_ref[...], preferred_element_type=jnp.float32)
    cls = cls + bc_ref[0]
    deltas = jnp.dot(shared, wd_ref[...], preferred_element_type=jnp.float32)
    deltas = deltas + bd_ref[0]

    i18 = jax.lax.broadcasted_iota(jnp.int32, (18, 9), 0)
    j9 = jax.lax.broadcasted_iota(jnp.int32, (18, 9), 1)
    e0 = (i18 == 2 * j9).astype(jnp.float32)
    e1 = (i18 == 2 * j9 + 1).astype(jnp.float32)
    s = jnp.dot(cls, e0, preferred_element_type=jnp.float32)
    t_ = jnp.dot(cls, e1, preferred_element_type=jnp.float32)
    p0 = jax.nn.sigmoid(s - t_)
    p1 = jax.nn.sigmoid(t_ - s)
    probs = (jnp.dot(p0, e0.T, preferred_element_type=jnp.float32)
             + jnp.dot(p1, e1.T, preferred_element_type=jnp.float32))

    for y in range(H):
        src = y * WP + 1
        probs_ref[0, y * W:(y + 1) * W, :] = probs[src:src + W, :]
        deltas_ref[0, y * W:(y + 1) * W, :] = deltas[src:src + W, :]


@jax.jit
def kernel(inputs, W_shared, b_shared, W_cls, b_cls, W_delta, b_delta):
    B = inputs.shape[0]
    nA = W_cls.shape[-1] // 2
    x = inputs.reshape(B, H * W, C)
    wf = W_shared.reshape(9, C, C)
    wc = W_cls.reshape(C, 2 * nA)
    wd = W_delta.reshape(C, 4 * nA)

    probs, deltas = pl.pallas_call(
        _rpn_kernel,
        grid=(B,),
        in_specs=[
            pl.BlockSpec((1, H * W, C), lambda i: (i, 0, 0)),
            pl.BlockSpec((9, C, C), lambda i: (0, 0, 0)),
            pl.BlockSpec((1, C), lambda i: (0, 0)),
            pl.BlockSpec((C, 2 * nA), lambda i: (0, 0)),
            pl.BlockSpec((1, 2 * nA), lambda i: (0, 0)),
            pl.BlockSpec((C, 4 * nA), lambda i: (0, 0)),
            pl.BlockSpec((1, 4 * nA), lambda i: (0, 0)),
        ],
        out_specs=[
            pl.BlockSpec((1, H * W, 2 * nA), lambda i: (i, 0, 0)),
            pl.BlockSpec((1, H * W, 4 * nA), lambda i: (i, 0, 0)),
        ],
        out_shape=[
            jax.ShapeDtypeStruct((B, H * W, 2 * nA), jnp.float32),
            jax.ShapeDtypeStruct((B, H * W, 4 * nA), jnp.float32),
        ],
        scratch_shapes=[pltpu.VMEM((FLAT, C), jnp.bfloat16)],
        compiler_params=pltpu.CompilerParams(
            dimension_semantics=("arbitrary",)),
    )(x, wf, b_shared.reshape(1, C), wc, b_cls.reshape(1, 2 * nA),
      wd, b_delta.reshape(1, 4 * nA))

    rpn_probs = probs.reshape(B, H * W * nA, 2)
    rpn_deltas = deltas.reshape(B, H * W * nA, 4)
    return (rpn_probs, rpn_deltas)


# speedup vs baseline: 1.0396x; 1.0396x over previous
import jax
import jax.numpy as jnp
from jax.experimental import pallas as pl
from jax.experimental.pallas import tpu as pltpu

H = 32
W = 32
C = 512
N = H * W
RS = H


def _shift_rows_down(a, k):
    return jnp.concatenate([jnp.zeros((k, a.shape[1]), a.dtype), a[:N - k]], axis=0)


def _shift_rows_up(a, k):
    return jnp.concatenate([a[k:], jnp.zeros((k, a.shape[1]), a.dtype)], axis=0)


def _rpn_kernel(x_ref, wf_ref, bs_ref, wc_ref, bc_ref, wd_ref, bd_ref,
                probs_ref, deltas_ref):
    x = x_ref[0]

    def col(c):
        a = jnp.dot(x, wf_ref[c], preferred_element_type=jnp.float32)
        b = jnp.dot(x, wf_ref[3 + c], preferred_element_type=jnp.float32)
        d = jnp.dot(x, wf_ref[6 + c], preferred_element_type=jnp.float32)
        return _shift_rows_down(a, RS) + b + _shift_rows_up(d, RS)

    t0 = col(0)
    t1 = col(1)
    t2 = col(2)

    r = jax.lax.broadcasted_iota(jnp.int32, (N, C), 0)
    in_row = r & (W - 1)
    left = jnp.where(in_row != 0, _shift_rows_down(t0, 1), 0.0)
    right = jnp.where(in_row != W - 1, _shift_rows_up(t2, 1), 0.0)
    shared = jnp.maximum(left + t1 + right + bs_ref[0], 0.0)

    cls = jnp.dot(shared, wc_ref[...], preferred_element_type=jnp.float32)
    cls = cls + bc_ref[0]
    deltas = jnp.dot(shared, wd_ref[...], preferred_element_type=jnp.float32)
    deltas_ref[0] = deltas + bd_ref[0]

    i18 = jax.lax.broadcasted_iota(jnp.int32, (18, 18), 0)
    j18 = jax.lax.broadcasted_iota(jnp.int32, (18, 18), 1)
    pm = (i18 == (j18 ^ 1)).astype(jnp.float32)
    swapped = jnp.dot(cls, pm.T, preferred_element_type=jnp.float32)
    probs_ref[0] = jax.nn.sigmoid(cls - swapped)


@jax.jit
def kernel(inputs, W_shared, b_shared, W_cls, b_cls, W_delta, b_delta):
    B = inputs.shape[0]
    nA = W_cls.shape[-1] // 2
    x = inputs.reshape(B, N, C)
    wf = W_shared.reshape(9, C, C)
    wc = W_cls.reshape(C, 2 * nA)
    wd = W_delta.reshape(C, 4 * nA)

    probs, deltas = pl.pallas_call(
        _rpn_kernel,
        grid=(B,),
        in_specs=[
            pl.BlockSpec((1, N, C), lambda i: (i, 0, 0)),
            pl.BlockSpec((9, C, C), lambda i: (0, 0, 0)),
            pl.BlockSpec((1, C), lambda i: (0, 0)),
            pl.BlockSpec((C, 2 * nA), lambda i: (0, 0)),
            pl.BlockSpec((1, 2 * nA), lambda i: (0, 0)),
            pl.BlockSpec((C, 4 * nA), lambda i: (0, 0)),
            pl.BlockSpec((1, 4 * nA), lambda i: (0, 0)),
        ],
        out_specs=[
            pl.BlockSpec((1, N, 2 * nA), lambda i: (i, 0, 0)),
            pl.BlockSpec((1, N, 4 * nA), lambda i: (i, 0, 0)),
        ],
        out_shape=[
            jax.ShapeDtypeStruct((B, N, 2 * nA), jnp.float32),
            jax.ShapeDtypeStruct((B, N, 4 * nA), jnp.float32),
        ],
        compiler_params=pltpu.CompilerParams(
            dimension_semantics=("arbitrary",)),
    )(x, wf, b_shared.reshape(1, C), wc, b_cls.reshape(1, 2 * nA),
      wd, b_delta.reshape(1, 4 * nA))

    rpn_probs = probs.reshape(B, N * nA, 2)
    rpn_deltas = deltas.reshape(B, N * nA, 4)
    return (rpn_probs, rpn_deltas)


# speedup vs baseline: 1.0403x; 1.0006x over previous
import jax
import jax.numpy as jnp
from jax.experimental import pallas as pl
from jax.experimental.pallas import tpu as pltpu

H = 32
W = 32
C = 512
N = H * W
RS = H


def _shift_rows_down(a, k):
    return jnp.concatenate([jnp.zeros((k, a.shape[1]), a.dtype), a[:N - k]], axis=0)


def _shift_rows_up(a, k):
    return jnp.concatenate([a[k:], jnp.zeros((k, a.shape[1]), a.dtype)], axis=0)


def _rpn_kernel(x_ref, wf_ref, bs_ref, wc_ref, bc_ref, wd_ref, bd_ref,
                probs_ref, deltas_ref):
    x = x_ref[0]

    def col(c):
        a = jnp.dot(x, wf_ref[c], preferred_element_type=jnp.float32)
        b = jnp.dot(x, wf_ref[3 + c], preferred_element_type=jnp.float32)
        d = jnp.dot(x, wf_ref[6 + c], preferred_element_type=jnp.float32)
        return _shift_rows_down(a, RS) + b + _shift_rows_up(d, RS)

    t0 = col(0)
    t1 = col(1)
    t2 = col(2)

    r = jax.lax.broadcasted_iota(jnp.int32, (N, C), 0)
    in_row = r & (W - 1)
    left = jnp.where(in_row != 0, _shift_rows_down(t0, 1), 0.0)
    right = jnp.where(in_row != W - 1, _shift_rows_up(t2, 1), 0.0)
    shared = jnp.maximum(left + t1 + right + bs_ref[0], 0.0)

    cls = jnp.dot(shared, wc_ref[...], preferred_element_type=jnp.float32)
    cls = cls + bc_ref[0]
    deltas = jnp.dot(shared, wd_ref[...], preferred_element_type=jnp.float32)
    deltas_ref[0] = deltas + bd_ref[0]

    i18 = jax.lax.broadcasted_iota(jnp.int32, (18, 18), 0)
    j18 = jax.lax.broadcasted_iota(jnp.int32, (18, 18), 1)
    pm = (i18 == (j18 ^ 1)).astype(jnp.float32)
    swapped = jnp.dot(cls, pm.T, preferred_element_type=jnp.float32)
    probs_ref[0] = jax.nn.sigmoid(cls - swapped)


@jax.jit
def kernel(inputs, W_shared, b_shared, W_cls, b_cls, W_delta, b_delta):
    B = inputs.shape[0]
    nA = W_cls.shape[-1] // 2
    x = inputs.reshape(B, N, C)
    wf = W_shared.reshape(9, C, C)
    wc = W_cls.reshape(C, 2 * nA)
    wd = W_delta.reshape(C, 4 * nA)

    probs, deltas = pl.pallas_call(
        _rpn_kernel,
        grid=(B,),
        in_specs=[
            pl.BlockSpec((1, N, C), lambda i: (i, 0, 0)),
            pl.BlockSpec((9, C, C), lambda i: (0, 0, 0)),
            pl.BlockSpec((1, C), lambda i: (0, 0)),
            pl.BlockSpec((C, 2 * nA), lambda i: (0, 0)),
            pl.BlockSpec((1, 2 * nA), lambda i: (0, 0)),
            pl.BlockSpec((C, 4 * nA), lambda i: (0, 0)),
            pl.BlockSpec((1, 4 * nA), lambda i: (0, 0)),
        ],
        out_specs=[
            pl.BlockSpec((1, N, 2 * nA), lambda i: (i, 0, 0)),
            pl.BlockSpec((1, N, 4 * nA), lambda i: (i, 0, 0)),
        ],
        out_shape=[
            jax.ShapeDtypeStruct((B, N, 2 * nA), jnp.float32),
            jax.ShapeDtypeStruct((B, N, 4 * nA), jnp.float32),
        ],
        compiler_params=pltpu.CompilerParams(
            dimension_semantics=("parallel",)),
    )(x, wf, b_shared.reshape(1, C), wc, b_cls.reshape(1, 2 * nA),
      wd, b_delta.reshape(1, 4 * nA))

    rpn_probs = probs.reshape(B, N * nA, 2)
    rpn_deltas = deltas.reshape(B, N * nA, 4)
    return (rpn_probs, rpn_deltas)


# speedup vs baseline: 1.0798x; 1.0380x over previous
import jax
import jax.numpy as jnp
from jax.experimental import pallas as pl

H = 32
W = 32
C = 512
WP = W + 2
HP = H + 4
FLAT = HP * WP
ROWS = H * WP
BASE = 2 * WP


def _rpn_kernel(x_ref, wf_ref, bs_ref, wc_ref, bc_ref, wd_ref, bd_ref,
                probs_ref, deltas_ref):
    x = x_ref[0]
    acc = jnp.zeros((ROWS, C), dtype=jnp.float32)
    for t in range(9):
        dy, dx = t // 3 - 1, t % 3 - 1
        start = BASE + dy * WP + dx
        acc = acc + jnp.dot(x[start:start + ROWS, :], wf_ref[t],
                            preferred_element_type=jnp.float32)
    shared = jnp.maximum(acc + bs_ref[0], 0.0)

    cls = jnp.dot(shared, wc_ref[...], preferred_element_type=jnp.float32)
    cls = cls + bc_ref[0]
    deltas = jnp.dot(shared, wd_ref[...], preferred_element_type=jnp.float32)
    deltas_ref[0] = deltas + bd_ref[0]

    i18 = jax.lax.broadcasted_iota(jnp.int32, (18, 9), 0)
    j9 = jax.lax.broadcasted_iota(jnp.int32, (18, 9), 1)
    e0 = (i18 == 2 * j9).astype(jnp.float32)
    e1 = (i18 == 2 * j9 + 1).astype(jnp.float32)
    s = jnp.dot(cls, e0, preferred_element_type=jnp.float32)
    t_ = jnp.dot(cls, e1, preferred_element_type=jnp.float32)
    p0 = jax.nn.sigmoid(s - t_)
    p1 = jax.nn.sigmoid(t_ - s)
    probs_ref[0] = (jnp.dot(p0, e0.T, preferred_element_type=jnp.float32)
                    + jnp.dot(p1, e1.T, preferred_element_type=jnp.float32))


@jax.jit
def kernel(inputs, W_shared, b_shared, W_cls, b_cls, W_delta, b_delta):
    B = inputs.shape[0]
    nA = W_cls.shape[-1] // 2
    xp = jnp.pad(inputs, ((0, 0), (2, 2), (1, 1), (0, 0)))
    xp = xp.reshape(B, FLAT, C)
    wf = W_shared.reshape(9, C, C)
    wc = W_cls.reshape(C, 2 * nA)
    wd = W_delta.reshape(C, 4 * nA)

    probs, deltas = pl.pallas_call(
        _rpn_kernel,
        grid=(B,),
        in_specs=[
            pl.BlockSpec((1, FLAT, C), lambda i: (i, 0, 0)),
            pl.BlockSpec((9, C, C), lambda i: (0, 0, 0)),
            pl.BlockSpec((1, C), lambda i: (0, 0)),
            pl.BlockSpec((C, 2 * nA), lambda i: (0, 0)),
            pl.BlockSpec((1, 2 * nA), lambda i: (0, 0)),
            pl.BlockSpec((C, 4 * nA), lambda i: (0, 0)),
            pl.BlockSpec((1, 4 * nA), lambda i: (0, 0)),
        ],
        out_specs=[
            pl.BlockSpec((1, ROWS, 2 * nA), lambda i: (i, 0, 0)),
            pl.BlockSpec((1, ROWS, 4 * nA), lambda i: (i, 0, 0)),
        ],
        out_shape=[
            jax.ShapeDtypeStruct((B, ROWS, 2 * nA), jnp.float32),
            jax.ShapeDtypeStruct((B, ROWS, 4 * nA), jnp.float32),
        ],
    )(xp, wf, b_shared.reshape(1, C), wc, b_cls.reshape(1, 2 * nA),
      wd, b_delta.reshape(1, 4 * nA))

    rpn_probs = probs.reshape(B, H, WP, nA, 2)[:, :, 1:W + 1]
    rpn_probs = rpn_probs.reshape(B, H * W * nA, 2)
    rpn_deltas = deltas.reshape(B, H, WP, nA, 4)[:, :, 1:W + 1]
    rpn_deltas = rpn_deltas.reshape(B, H * W * nA, 4)
    return (rpn_probs, rpn_deltas)
